# baseline (speedup 1.0000x reference)
import functools
import jax
import jax.numpy as jnp
from jax import lax
from jax.experimental import pallas as pl
from jax.experimental.pallas import tpu as pltpu
from jax.experimental.pallas import tpu_sc as plsc

NC = 2
NS = 16
CH = 160
W = 16


def _row(b):
    return b.reshape(1, -1)


def _pack16(h):
    u = lax.bitcast_convert_type(h.astype(jnp.bfloat16), jnp.uint16)
    half = u.shape[1] // 2
    lo = u[:, :half].astype(jnp.int32)
    hi = u[:, half:].astype(jnp.int32)
    return lo | (hi << 16)


def _unpack16(w):
    lo = lax.bitcast_convert_type(w << 16, jnp.float32)
    hi = lax.bitcast_convert_type((w >> 16) << 16, jnp.float32)
    return jnp.concatenate([lo, hi], axis=1)


def _sel_inv(cnt):
    col = lax.broadcasted_iota(jnp.int32, cnt.shape, 1)
    sel = ((col == 4) | (col == 5)).astype(jnp.float32)
    return sel / (cnt + 1e-5)


def _mlp_edge_body(x_ref, w1_ref, b1_ref, w2_ref, b2_ref, wg_ref, bg_ref,
                   h_ref, es_ref):
    h1 = jnp.maximum(x_ref[...] @ w1_ref[...] + b1_ref[...], 0.0)
    h = h1 @ w2_ref[...] + b2_ref[...]
    h_ref[...] = h
    es_ref[...] = h @ wg_ref[...] + bg_ref[...]


def _mlp_node_body(x_ref, w1_ref, b1_ref, w2_ref, b2_ref, wg_ref, cnt_ref,
                   h_ref, tab_ref):
    h1 = jnp.maximum(x_ref[...] @ w1_ref[...] + b1_ref[...], 0.0)
    h = h1 @ w2_ref[...] + b2_ref[...]
    h_ref[...] = h
    tab_ref[...] = h @ wg_ref[...] + _sel_inv(cnt_ref[...])


def _gates_body(ts_ref, to_ref, es_ref, w8_ref):
    ts = ts_ref[...]
    to = to_ref[...]
    es = es_ref[...]
    sig = jax.nn.sigmoid
    w_s = 0.5 * sig(ts[:, 0:1] + es[:, 0:1]) * ts[:, 4:5]
    w_o = 0.5 * sig(to[:, 1:2] + es[:, 1:2]) * to[:, 5:6]
    gsp = 0.5 * sig(ts[:, 2:3] + es[:, 2:3])
    gop = 0.5 * sig(to[:, 3:4] + es[:, 3:4])
    z = jnp.zeros((ts.shape[0], W - 4), jnp.float32)
    w8_ref[...] = jnp.concatenate([w_s, w_o, gsp, gop, z], axis=1)


def _gru(x, h, wih, whh, bih, bhh):
    d = h.shape[1]
    gi = x @ wih + bih
    gh = h @ whh + bhh
    r = jax.nn.sigmoid(gi[:, :d] + gh[:, :d])
    z = jax.nn.sigmoid(gi[:, d:2 * d] + gh[:, d:2 * d])
    n = jnp.tanh(gi[:, 2 * d:] + r * gh[:, 2 * d:])
    return (1.0 - z) * n + z * h


def _edge_step_body(h_ref, sub_ref, obj_ref, w8_ref, wih_ref, whh_ref,
                    bih_ref, bhh_ref, wg_ref, bg_ref, wc_ref, bc_ref,
                    out_ref, es_ref, logits_ref):
    h = h_ref[...]
    w8 = w8_ref[...]
    msg = w8[:, 2:3] * sub_ref[...] + w8[:, 3:4] * obj_ref[...]
    new = _gru(msg, h, wih_ref[...], whh_ref[...], bih_ref[...], bhh_ref[...])
    out_ref[...] = new
    es_ref[...] = new @ wg_ref[...] + bg_ref[...]
    logits_ref[...] = new @ wc_ref[...] + bc_ref[...]


def _node_step_body(msg_ref, h_ref, wih_ref, whh_ref, bih_ref, bhh_ref,
                    wg_ref, cnt_ref, wc_ref, bc_ref,
                    out_ref, tab_ref, logits_ref, labels_ref):
    new = _gru(msg_ref[...], h_ref[...], wih_ref[...], whh_ref[...],
               bih_ref[...], bhh_ref[...])
    out_ref[...] = new
    tab_ref[...] = new @ wg_ref[...] + _sel_inv(cnt_ref[...])
    logits = new @ wc_ref[...] + bc_ref[...]
    logits_ref[...] = logits
    lab = jnp.argmax(logits[:, 1:], axis=1).astype(jnp.int32) + 1
    labels_ref[...] = jnp.broadcast_to(lab[:, None], labels_ref.shape)


def _full(shape):
    return pl.BlockSpec(shape, lambda i: (0, 0))


def _rows(blk, width):
    return pl.BlockSpec((blk, width), lambda i: (i, 0))


def _mlp_edge(x, w1, b1, w2, b2, wg, bg, blk=1000):
    m, _ = x.shape
    d = w2.shape[1]
    return pl.pallas_call(
        _mlp_edge_body,
        grid=(m // blk,),
        in_specs=[_rows(blk, x.shape[1]), _full(w1.shape), _full(b1.shape),
                  _full(w2.shape), _full(b2.shape), _full(wg.shape),
                  _full(bg.shape)],
        out_specs=[_rows(blk, d), _rows(blk, W)],
        out_shape=[jax.ShapeDtypeStruct((m, d), jnp.float32),
                   jax.ShapeDtypeStruct((m, W), jnp.float32)],
    )(x, w1, b1, w2, b2, wg, bg)


def _mlp_node(x, w1, b1, w2, b2, wg, cnt, blk=1000):
    m, _ = x.shape
    d = w2.shape[1]
    return pl.pallas_call(
        _mlp_node_body,
        grid=(m // blk,),
        in_specs=[_rows(blk, x.shape[1]), _full(w1.shape), _full(b1.shape),
                  _full(w2.shape), _full(b2.shape), _full(wg.shape),
                  _rows(blk, 128)],
        out_specs=[_rows(blk, d), _rows(blk, 128)],
        out_shape=[jax.ShapeDtypeStruct((m, d), jnp.float32),
                   jax.ShapeDtypeStruct((m, 128), jnp.float32)],
    )(x, w1, b1, w2, b2, wg, cnt)


def _gates(ts, to, es, blk=2000):
    m = ts.shape[0]
    return pl.pallas_call(
        _gates_body,
        grid=(m // blk,),
        in_specs=[_rows(blk, 128), _rows(blk, 128), _rows(blk, W)],
        out_specs=_rows(blk, W),
        out_shape=jax.ShapeDtypeStruct((m, W), jnp.float32),
    )(ts, to, es)


def _edge_step(h, sub, obj, w8, wih, whh, bih, bhh, wg, bg, wc, bc, blk=1000):
    m, d = h.shape
    nc = wc.shape[1]
    return pl.pallas_call(
        _edge_step_body,
        grid=(m // blk,),
        in_specs=[_rows(blk, d), _rows(blk, d), _rows(blk, d),
                  _rows(blk, W),
                  _full(wih.shape), _full(whh.shape), _full(bih.shape),
                  _full(bhh.shape), _full(wg.shape), _full(bg.shape),
                  _full(wc.shape), _full(bc.shape)],
        out_specs=[_rows(blk, d), _rows(blk, W), _rows(blk, nc)],
        out_shape=[jax.ShapeDtypeStruct((m, d), jnp.float32),
                   jax.ShapeDtypeStruct((m, W), jnp.float32),
                   jax.ShapeDtypeStruct((m, nc), jnp.float32)],
    )(h, sub, obj, w8, wih, whh, bih, bhh, wg, bg, wc, bc)


def _node_step(msg, h, wih, whh, bih, bhh, wg, cnt, wc, bc, blk=1000):
    m, d = h.shape
    nc = wc.shape[1]
    return pl.pallas_call(
        _node_step_body,
        grid=(m // blk,),
        in_specs=[_rows(blk, d), _rows(blk, d),
                  _full(wih.shape), _full(whh.shape), _full(bih.shape),
                  _full(bhh.shape), _full(wg.shape), _rows(blk, 128),
                  _full(wc.shape), _full(bc.shape)],
        out_specs=[_rows(blk, d), _rows(blk, 128),
                   _rows(blk, nc), _rows(blk, 8)],
        out_shape=[jax.ShapeDtypeStruct((m, d), jnp.float32),
                   jax.ShapeDtypeStruct((m, 128), jnp.float32),
                   jax.ShapeDtypeStruct((m, nc), jnp.float32),
                   jax.ShapeDtypeStruct((m, 8), jnp.int32)],
    )(msg, h, wih, whh, bih, bhh, wg, cnt, wc, bc)


def _sc_mesh():
    return plsc.VectorSubcoreMesh(core_axis_name="c", subcore_axis_name="s")


def _sc_count(subj, objn, n):
    e = subj.shape[0]
    nch = e // CH

    @functools.partial(
        pl.kernel,
        out_type=jax.ShapeDtypeStruct((n, 128), jnp.float32),
        mesh=_sc_mesh(),
        scratch_types=[
            pltpu.VMEM((CH,), jnp.int32),
            pltpu.VMEM((CH, 128), jnp.float32),
            pltpu.VMEM((CH, 128), jnp.float32),
            pltpu.VMEM_SHARED((n, 128), jnp.float32),
            pltpu.SemaphoreType.DMA,
        ],
    )
    def k(subj_ref, objn_ref, zero_ref, u4_ref, u5_ref, out_ref, idxb, ub4,
          ub5, acc, sem):
        c = lax.axis_index("c")
        s = lax.axis_index("s")
        rows = n // NS

        pltpu.sync_copy(u4_ref, ub4)
        pltpu.sync_copy(u5_ref, ub5)

        @pl.when(c == 0)
        def _():
            pltpu.sync_copy(zero_ref.at[pl.ds(0, rows), :],
                            acc.at[pl.ds(s * rows, rows), :])
            @pl.when(s == 0)
            def _():
                pltpu.sync_copy(zero_ref.at[pl.ds(0, n - rows * NS), :],
                                acc.at[pl.ds(rows * NS, n - rows * NS), :])
        plsc.subcore_barrier()

        @pl.when(c == 0)
        def _():
            def body(k_, _):
                e0 = (s + k_ * NS) * CH
                pltpu.sync_copy(subj_ref.at[pl.ds(e0, CH)], idxb)
                pltpu.sync_copy(ub4, acc.at[idxb], add=True)
                pltpu.sync_copy(objn_ref.at[pl.ds(e0, CH)], idxb)
                pltpu.sync_copy(ub5, acc.at[idxb], add=True)
                return 0

            trip = (nch - s + NS - 1) // NS
            lax.fori_loop(0, trip, body, 0)
        plsc.subcore_barrier()

        @pl.when(c == 0)
        def _():
            pltpu.sync_copy(acc.at[pl.ds(s * rows, rows), :],
                            out_ref.at[pl.ds(s * rows, rows), :])
            @pl.when(s == 0)
            def _():
                pltpu.sync_copy(acc.at[pl.ds(rows * NS, n - rows * NS), :],
                                out_ref.at[pl.ds(rows * NS, n - rows * NS), :])

    zeros = jnp.zeros((n // NS, 128), jnp.float32)
    col = jnp.arange(128)
    u4 = jnp.broadcast_to((col == 4).astype(jnp.float32), (CH, 128))
    u5 = jnp.broadcast_to((col == 5).astype(jnp.float32), (CH, 128))
    return k(subj, objn, zeros, u4, u5)


def _sc_gather(hxp, tab, subj, objn):
    n, dp = hxp.shape
    e = subj.shape[0]
    chg = 80
    nch = e // chg
    nw = NC * NS

    @functools.partial(
        pl.kernel,
        out_type=[jax.ShapeDtypeStruct((e, dp), jnp.float32),
                  jax.ShapeDtypeStruct((e, dp), jnp.float32),
                  jax.ShapeDtypeStruct((e, 128), jnp.float32),
                  jax.ShapeDtypeStruct((e, 128), jnp.float32)],
        mesh=_sc_mesh(),
        scratch_types=[
            pltpu.VMEM((chg,), jnp.int32),
            pltpu.VMEM((chg,), jnp.int32),
            pltpu.VMEM((chg, dp), jnp.float32),
            pltpu.VMEM((chg, dp), jnp.float32),
            pltpu.VMEM((chg, 128), jnp.float32),
            pltpu.VMEM((chg, 128), jnp.float32),
            pltpu.SemaphoreType.DMA,
            pltpu.SemaphoreType.DMA,
            pltpu.SemaphoreType.DMA,
            pltpu.SemaphoreType.DMA,
            pltpu.SemaphoreType.DMA,
            pltpu.SemaphoreType.DMA,
        ],
    )
    def k(hx_ref, tab_ref, subj_ref, objn_ref, sub_out, obj_out, ts_out,
          to_out, idxs, idxo, subb, objb, tsb, tob, sem, sga, sgb, sgc,
          sgd, semw):
        c = lax.axis_index("c")
        s = lax.axis_index("s")
        wid = s * NC + c

        def body(k_, _):
            e0 = (wid + k_ * nw) * chg
            c1 = pltpu.async_copy(subj_ref.at[pl.ds(e0, chg)], idxs, sem)
            c2 = pltpu.async_copy(objn_ref.at[pl.ds(e0, chg)], idxo, sem)
            c1.wait()
            c2.wait()
            g1 = pltpu.async_copy(hx_ref.at[idxs], subb, sga)
            g2 = pltpu.async_copy(hx_ref.at[idxo], objb, sgb)
            g3 = pltpu.async_copy(tab_ref.at[idxs], tsb, sgc)
            g4 = pltpu.async_copy(tab_ref.at[idxo], tob, sgd)
            g1.wait()
            w1 = pltpu.async_copy(subb, sub_out.at[pl.ds(e0, chg), :], semw)
            g2.wait()
            w2 = pltpu.async_copy(objb, obj_out.at[pl.ds(e0, chg), :], semw)
            g3.wait()
            w3 = pltpu.async_copy(tsb, ts_out.at[pl.ds(e0, chg), :], semw)
            g4.wait()
            w4 = pltpu.async_copy(tob, to_out.at[pl.ds(e0, chg), :], semw)
            w1.wait()
            w2.wait()
            w3.wait()
            w4.wait()
            return 0

        trip = (nch - wid + nw - 1) // nw
        lax.fori_loop(0, trip, body, 0)

    return k(hxp, tab, subj, objn)


def _sc_scatter(hx_edge, w8, subj, objn, n):
    e, d = hx_edge.shape
    dh = 128
    nch = e // CH
    rows = n // NS

    @functools.partial(
        pl.kernel,
        out_type=jax.ShapeDtypeStruct((n, d), jnp.float32),
        mesh=_sc_mesh(),
        scratch_types=[
            pltpu.VMEM((CH,), jnp.int32),
            pltpu.VMEM((CH,), jnp.int32),
            pltpu.VMEM((CH, W), jnp.float32),
            pltpu.VMEM((CH, dh), jnp.float32),
            pltpu.VMEM((CH, dh), jnp.float32),
            pltpu.VMEM((CH, dh), jnp.float32),
            pltpu.VMEM_SHARED((n, dh), jnp.float32),
            pltpu.SemaphoreType.DMA,
            pltpu.SemaphoreType.DMA,
        ],
    )
    def k(hx_ref, w8_ref, subj_ref, objn_ref, zero_ref, out_ref,
          idxs, idxo, wb, buf, ms, mo, acc, sem, sem2):
        c = lax.axis_index("c")
        s = lax.axis_index("s")

        for p in range(2):
            c0 = c * 256 + p * dh
            _scatter_pass(hx_ref, w8_ref, subj_ref, objn_ref, zero_ref,
                          out_ref, idxs, idxo, wb, buf, ms, mo, acc, sem,
                          sem2, s, c0, n, nch, rows, dh)

    zeros = jnp.zeros((n // NS, dh), jnp.float32)
    return k(hx_edge, w8, subj, objn, zeros)


def _scatter_pass(hx_ref, w8_ref, subj_ref, objn_ref, zero_ref, out_ref,
                  idxs, idxo, wb, buf, ms, mo, acc, sem, sem2, s, c0, n,
                  nch, rows, dh):
        pltpu.sync_copy(zero_ref.at[pl.ds(0, rows), :],
                        acc.at[pl.ds(s * rows, rows), :])
        @pl.when(s == 0)
        def _():
            pltpu.sync_copy(zero_ref.at[pl.ds(0, n - rows * NS), :],
                            acc.at[pl.ds(rows * NS, n - rows * NS), :])
        plsc.subcore_barrier()

        def body(k_, _):
            e0 = (s + k_ * NS) * CH

            @pl.when(k_ > 0)
            def _():
                pltpu.make_async_copy(ms, acc.at[idxs], sem2).wait()
                pltpu.make_async_copy(mo, acc.at[idxo], sem2).wait()

            l1 = pltpu.async_copy(subj_ref.at[pl.ds(e0, CH)], idxs, sem)
            l2 = pltpu.async_copy(objn_ref.at[pl.ds(e0, CH)], idxo, sem)
            l3 = pltpu.async_copy(w8_ref.at[pl.ds(e0, CH), :], wb, sem)
            l4 = pltpu.async_copy(
                hx_ref.at[pl.ds(e0, CH), pl.ds(c0, dh)], buf, sem)
            l1.wait()
            l2.wait()
            l3.wait()
            l4.wait()

            def scale(j, _):
                wv = wb[j, :]
                ws = wv[0]
                wo = wv[1]
                for t in range(dh // 16):
                    v = buf[j, pl.ds(t * 16, 16)]
                    ms[j, pl.ds(t * 16, 16)] = v * ws
                    mo[j, pl.ds(t * 16, 16)] = v * wo
                return 0

            lax.fori_loop(0, CH, scale, 0)
            pltpu.async_copy(ms, acc.at[idxs], sem2, add=True)
            pltpu.async_copy(mo, acc.at[idxo], sem2, add=True)
            return 0

        trip = (nch - s + NS - 1) // NS
        lax.fori_loop(0, trip, body, 0)
        pltpu.make_async_copy(ms, acc.at[idxs], sem2).wait()
        pltpu.make_async_copy(mo, acc.at[idxo], sem2).wait()
        plsc.subcore_barrier()

        pltpu.sync_copy(acc.at[pl.ds(s * rows, rows), :],
                        out_ref.at[pl.ds(s * rows, rows), pl.ds(c0, dh)])
        @pl.when(s == 0)
        def _():
            pltpu.sync_copy(
                acc.at[pl.ds(rows * NS, n - rows * NS), :],
                out_ref.at[pl.ds(rows * NS, n - rows * NS), pl.ds(c0, dh)])
        plsc.subcore_barrier()


def kernel(x_obj, x_pred_feat, rel_inds, obj_W1, obj_b1, obj_W2, obj_b2,
           pred_W1, pred_b1, pred_W2, pred_b2, node_Wih, node_Whh, node_bih,
           node_bhh, edge_Wih, edge_Whh, edge_bih, edge_bhh, sn_W, sn_b,
           on_W, on_b, se_W, se_b, oe_W, oe_b, objcls_W, objcls_b, relcls_W,
           relcls_b):
    n = x_obj.shape[0]
    e = x_pred_feat.shape[0]
    d = obj_W2.shape[1]
    subj = rel_inds[:, 0]
    objn = rel_inds[:, 1]

    cnt = _sc_count(subj, objn, n)

    wg_n = jnp.concatenate(
        [sn_W[:d], on_W[:d], se_W[:d], oe_W[:d],
         jnp.zeros((d, 124), jnp.float32)], axis=1)
    wg_e = jnp.concatenate(
        [sn_W[d:], on_W[d:], se_W[d:], oe_W[d:],
         jnp.zeros((d, W - 4), jnp.float32)], axis=1)
    bg_e = jnp.concatenate([sn_b, on_b, se_b, oe_b,
                            jnp.zeros((W - 4,), jnp.float32)]).reshape(1, W)

    hx_edge, es = _mlp_edge(x_pred_feat, pred_W1, _row(pred_b1), pred_W2,
                            _row(pred_b2), wg_e, bg_e)
    hx_obj, tab = _mlp_node(x_obj, obj_W1, _row(obj_b1), obj_W2,
                            _row(obj_b2), wg_n, cnt)

    nbih, nbhh = _row(node_bih), _row(node_bhh)
    ebih, ebhh = _row(edge_bih), _row(edge_bhh)
    ocb, rcb = _row(objcls_b), _row(relcls_b)

    for _ in range(2):
        sub, obj, ts, to = _sc_gather(hx_obj, tab, subj, objn)
        w8 = _gates(ts, to, es)
        msg = _sc_scatter(hx_edge, w8, subj, objn, n)
        hx_obj, tab, obj_logits, labels8 = _node_step(
            msg, hx_obj, node_Wih, node_Whh, nbih, nbhh, wg_n, cnt,
            objcls_W, ocb)
        hx_edge, es, rel_logits = _edge_step(
            hx_edge, sub, obj, w8, edge_Wih, edge_Whh, ebih, ebhh, wg_e,
            bg_e, relcls_W, rcb)

    labels = labels8[:, 0]
    return ((hx_obj, hx_edge), obj_logits, rel_logits, labels, rel_inds)


# speedup vs baseline: 1.0023x; 1.0023x over previous
import functools
import jax
import jax.numpy as jnp
from jax import lax
from jax.experimental import pallas as pl
from jax.experimental.pallas import tpu as pltpu
from jax.experimental.pallas import tpu_sc as plsc

NC = 2
NS = 16
CH = 160
W = 16


def _row(b):
    return b.reshape(1, -1)


def _sel_inv(cnt):
    col = lax.broadcasted_iota(jnp.int32, cnt.shape, 1)
    sel = ((col == 4) | (col == 5)).astype(jnp.float32)
    return sel / (cnt + 1e-5)


def _mlp_edge_body(x_ref, w1_ref, b1_ref, w2_ref, b2_ref, wg_ref, bg_ref,
                   h_ref, es_ref):
    h1 = jnp.maximum(x_ref[...] @ w1_ref[...] + b1_ref[...], 0.0)
    h = h1 @ w2_ref[...] + b2_ref[...]
    h_ref[...] = h
    es_ref[...] = h @ wg_ref[...] + bg_ref[...]


def _mlp_node_body(x_ref, w1_ref, b1_ref, w2_ref, b2_ref, wg_ref, cnt_ref,
                   h_ref, tab_ref):
    h1 = jnp.maximum(x_ref[...] @ w1_ref[...] + b1_ref[...], 0.0)
    h = h1 @ w2_ref[...] + b2_ref[...]
    h_ref[...] = h
    tab_ref[...] = h @ wg_ref[...] + _sel_inv(cnt_ref[...])


def _gates_body(ts_ref, to_ref, es_ref, w8_ref):
    ts = ts_ref[...]
    to = to_ref[...]
    es = es_ref[...]
    sig = jax.nn.sigmoid
    w_s = 0.5 * sig(ts[:, 0:1] + es[:, 0:1]) * ts[:, 4:5]
    w_o = 0.5 * sig(to[:, 1:2] + es[:, 1:2]) * to[:, 5:6]
    gsp = 0.5 * sig(ts[:, 2:3] + es[:, 2:3])
    gop = 0.5 * sig(to[:, 3:4] + es[:, 3:4])
    z = jnp.zeros((ts.shape[0], W - 4), jnp.float32)
    w8_ref[...] = jnp.concatenate([w_s, w_o, gsp, gop, z], axis=1)


def _gru(x, h, wih, whh, bih, bhh):
    d = h.shape[1]
    gi = x @ wih + bih
    gh = h @ whh + bhh
    r = jax.nn.sigmoid(gi[:, :d] + gh[:, :d])
    z = jax.nn.sigmoid(gi[:, d:2 * d] + gh[:, d:2 * d])
    n = jnp.tanh(gi[:, 2 * d:] + r * gh[:, 2 * d:])
    return (1.0 - z) * n + z * h


def _edge_step_body(h_ref, sub_ref, obj_ref, w8_ref, wih_ref, whh_ref,
                    bih_ref, bhh_ref, wg_ref, bg_ref, wc_ref, bc_ref,
                    out_ref, es_ref, logits_ref):
    h = h_ref[...]
    w8 = w8_ref[...]
    msg = w8[:, 2:3] * sub_ref[...] + w8[:, 3:4] * obj_ref[...]
    new = _gru(msg, h, wih_ref[...], whh_ref[...], bih_ref[...], bhh_ref[...])
    out_ref[...] = new
    es_ref[...] = new @ wg_ref[...] + bg_ref[...]
    logits_ref[...] = new @ wc_ref[...] + bc_ref[...]


def _node_step_body(msg_ref, h_ref, wih_ref, whh_ref, bih_ref, bhh_ref,
                    wg_ref, cnt_ref, wc_ref, bc_ref,
                    out_ref, tab_ref, logits_ref, labels_ref):
    new = _gru(msg_ref[...], h_ref[...], wih_ref[...], whh_ref[...],
               bih_ref[...], bhh_ref[...])
    out_ref[...] = new
    tab_ref[...] = new @ wg_ref[...] + _sel_inv(cnt_ref[...])
    logits = new @ wc_ref[...] + bc_ref[...]
    logits_ref[...] = logits
    lab = jnp.argmax(logits[:, 1:], axis=1).astype(jnp.int32) + 1
    labels_ref[...] = jnp.broadcast_to(lab[:, None], labels_ref.shape)


def _full(shape):
    return pl.BlockSpec(shape, lambda i: (0, 0))


def _rows(blk, width):
    return pl.BlockSpec((blk, width), lambda i: (i, 0))


def _mlp_edge(x, w1, b1, w2, b2, wg, bg, blk=1000):
    m, _ = x.shape
    d = w2.shape[1]
    return pl.pallas_call(
        _mlp_edge_body,
        grid=(m // blk,),
        in_specs=[_rows(blk, x.shape[1]), _full(w1.shape), _full(b1.shape),
                  _full(w2.shape), _full(b2.shape), _full(wg.shape),
                  _full(bg.shape)],
        out_specs=[_rows(blk, d), _rows(blk, W)],
        out_shape=[jax.ShapeDtypeStruct((m, d), jnp.float32),
                   jax.ShapeDtypeStruct((m, W), jnp.float32)],
    )(x, w1, b1, w2, b2, wg, bg)


def _mlp_node(x, w1, b1, w2, b2, wg, cnt, blk=1000):
    m, _ = x.shape
    d = w2.shape[1]
    return pl.pallas_call(
        _mlp_node_body,
        grid=(m // blk,),
        in_specs=[_rows(blk, x.shape[1]), _full(w1.shape), _full(b1.shape),
                  _full(w2.shape), _full(b2.shape), _full(wg.shape),
                  _rows(blk, 128)],
        out_specs=[_rows(blk, d), _rows(blk, 128)],
        out_shape=[jax.ShapeDtypeStruct((m, d), jnp.float32),
                   jax.ShapeDtypeStruct((m, 128), jnp.float32)],
    )(x, w1, b1, w2, b2, wg, cnt)


def _gates(ts, to, es, blk=2000):
    m = ts.shape[0]
    return pl.pallas_call(
        _gates_body,
        grid=(m // blk,),
        in_specs=[_rows(blk, 128), _rows(blk, 128), _rows(blk, W)],
        out_specs=_rows(blk, W),
        out_shape=jax.ShapeDtypeStruct((m, W), jnp.float32),
    )(ts, to, es)


def _edge_step(h, sub, obj, w8, wih, whh, bih, bhh, wg, bg, wc, bc, blk=1000):
    m, d = h.shape
    nc = wc.shape[1]
    return pl.pallas_call(
        _edge_step_body,
        grid=(m // blk,),
        in_specs=[_rows(blk, d), _rows(blk, d), _rows(blk, d),
                  _rows(blk, W),
                  _full(wih.shape), _full(whh.shape), _full(bih.shape),
                  _full(bhh.shape), _full(wg.shape), _full(bg.shape),
                  _full(wc.shape), _full(bc.shape)],
        out_specs=[_rows(blk, d), _rows(blk, W), _rows(blk, nc)],
        out_shape=[jax.ShapeDtypeStruct((m, d), jnp.float32),
                   jax.ShapeDtypeStruct((m, W), jnp.float32),
                   jax.ShapeDtypeStruct((m, nc), jnp.float32)],
    )(h, sub, obj, w8, wih, whh, bih, bhh, wg, bg, wc, bc)


def _node_step(msg, h, wih, whh, bih, bhh, wg, cnt, wc, bc, blk=1000):
    m, d = h.shape
    nc = wc.shape[1]
    return pl.pallas_call(
        _node_step_body,
        grid=(m // blk,),
        in_specs=[_rows(blk, d), _rows(blk, d),
                  _full(wih.shape), _full(whh.shape), _full(bih.shape),
                  _full(bhh.shape), _full(wg.shape), _rows(blk, 128),
                  _full(wc.shape), _full(bc.shape)],
        out_specs=[_rows(blk, d), _rows(blk, 128),
                   _rows(blk, nc), _rows(blk, 8)],
        out_shape=[jax.ShapeDtypeStruct((m, d), jnp.float32),
                   jax.ShapeDtypeStruct((m, 128), jnp.float32),
                   jax.ShapeDtypeStruct((m, nc), jnp.float32),
                   jax.ShapeDtypeStruct((m, 8), jnp.int32)],
    )(msg, h, wih, whh, bih, bhh, wg, cnt, wc, bc)


def _sc_mesh():
    return plsc.VectorSubcoreMesh(core_axis_name="c", subcore_axis_name="s")


def _sc_count(subj, objn, n):
    e = subj.shape[0]
    nch = e // CH

    @functools.partial(
        pl.kernel,
        out_type=jax.ShapeDtypeStruct((n, 128), jnp.float32),
        mesh=_sc_mesh(),
        scratch_types=[
            pltpu.VMEM((CH,), jnp.int32),
            pltpu.VMEM((CH, 128), jnp.float32),
            pltpu.VMEM((CH, 128), jnp.float32),
            pltpu.VMEM_SHARED((n, 128), jnp.float32),
            pltpu.SemaphoreType.DMA,
        ],
    )
    def k(subj_ref, objn_ref, zero_ref, u4_ref, u5_ref, out_ref, idxb, ub4,
          ub5, acc, sem):
        c = lax.axis_index("c")
        s = lax.axis_index("s")
        rows = n // NS

        pltpu.sync_copy(u4_ref, ub4)
        pltpu.sync_copy(u5_ref, ub5)

        @pl.when(c == 0)
        def _():
            pltpu.sync_copy(zero_ref.at[pl.ds(0, rows), :],
                            acc.at[pl.ds(s * rows, rows), :])
            @pl.when(s == 0)
            def _():
                pltpu.sync_copy(zero_ref.at[pl.ds(0, n - rows * NS), :],
                                acc.at[pl.ds(rows * NS, n - rows * NS), :])
        plsc.subcore_barrier()

        @pl.when(c == 0)
        def _():
            def body(k_, _):
                e0 = (s + k_ * NS) * CH
                pltpu.sync_copy(subj_ref.at[pl.ds(e0, CH)], idxb)
                pltpu.sync_copy(ub4, acc.at[idxb], add=True)
                pltpu.sync_copy(objn_ref.at[pl.ds(e0, CH)], idxb)
                pltpu.sync_copy(ub5, acc.at[idxb], add=True)
                return 0

            trip = (nch - s + NS - 1) // NS
            lax.fori_loop(0, trip, body, 0)
        plsc.subcore_barrier()

        @pl.when(c == 0)
        def _():
            pltpu.sync_copy(acc.at[pl.ds(s * rows, rows), :],
                            out_ref.at[pl.ds(s * rows, rows), :])
            @pl.when(s == 0)
            def _():
                pltpu.sync_copy(acc.at[pl.ds(rows * NS, n - rows * NS), :],
                                out_ref.at[pl.ds(rows * NS, n - rows * NS), :])

    zeros = jnp.zeros((n // NS, 128), jnp.float32)
    col = jnp.arange(128)
    u4 = jnp.broadcast_to((col == 4).astype(jnp.float32), (CH, 128))
    u5 = jnp.broadcast_to((col == 5).astype(jnp.float32), (CH, 128))
    return k(subj, objn, zeros, u4, u5)


def _sc_gather(hxp, tab, subj, objn):
    n, dp = hxp.shape
    e = subj.shape[0]
    chg = 80
    nch = e // chg
    nw = NC * NS

    @functools.partial(
        pl.kernel,
        out_type=[jax.ShapeDtypeStruct((e, dp), jnp.float32),
                  jax.ShapeDtypeStruct((e, dp), jnp.float32),
                  jax.ShapeDtypeStruct((e, 128), jnp.float32),
                  jax.ShapeDtypeStruct((e, 128), jnp.float32)],
        mesh=_sc_mesh(),
        scratch_types=[
            pltpu.VMEM((chg,), jnp.int32),
            pltpu.VMEM((chg,), jnp.int32),
            pltpu.VMEM((chg, dp), jnp.float32),
            pltpu.VMEM((chg, dp), jnp.float32),
            pltpu.VMEM((chg, 128), jnp.float32),
            pltpu.VMEM((chg, 128), jnp.float32),
            pltpu.SemaphoreType.DMA,
            pltpu.SemaphoreType.DMA,
            pltpu.SemaphoreType.DMA,
            pltpu.SemaphoreType.DMA,
            pltpu.SemaphoreType.DMA,
            pltpu.SemaphoreType.DMA,
        ],
    )
    def k(hx_ref, tab_ref, subj_ref, objn_ref, sub_out, obj_out, ts_out,
          to_out, idxs, idxo, subb, objb, tsb, tob, sem, sga, sgb, sgc,
          sgd, semw):
        c = lax.axis_index("c")
        s = lax.axis_index("s")
        wid = s * NC + c

        def body(k_, _):
            e0 = (wid + k_ * nw) * chg
            c1 = pltpu.async_copy(subj_ref.at[pl.ds(e0, chg)], idxs, sem)
            c2 = pltpu.async_copy(objn_ref.at[pl.ds(e0, chg)], idxo, sem)
            c1.wait()
            c2.wait()
            g1 = pltpu.async_copy(hx_ref.at[idxs], subb, sga)
            g2 = pltpu.async_copy(hx_ref.at[idxo], objb, sgb)
            g3 = pltpu.async_copy(tab_ref.at[idxs], tsb, sgc)
            g4 = pltpu.async_copy(tab_ref.at[idxo], tob, sgd)
            g1.wait()
            w1 = pltpu.async_copy(subb, sub_out.at[pl.ds(e0, chg), :], semw)
            g2.wait()
            w2 = pltpu.async_copy(objb, obj_out.at[pl.ds(e0, chg), :], semw)
            g3.wait()
            w3 = pltpu.async_copy(tsb, ts_out.at[pl.ds(e0, chg), :], semw)
            g4.wait()
            w4 = pltpu.async_copy(tob, to_out.at[pl.ds(e0, chg), :], semw)
            w1.wait()
            w2.wait()
            w3.wait()
            w4.wait()
            return 0

        trip = (nch - wid + nw - 1) // nw
        lax.fori_loop(0, trip, body, 0)

    return k(hxp, tab, subj, objn)


def _sc_scatter(hx_edge, w8, subj, objn, n):
    e, d = hx_edge.shape
    dh = 128
    nch = e // CH
    rows = n // NS

    @functools.partial(
        pl.kernel,
        out_type=jax.ShapeDtypeStruct((n, d), jnp.float32),
        mesh=_sc_mesh(),
        scratch_types=[
            pltpu.VMEM((CH,), jnp.int32),
            pltpu.VMEM((CH,), jnp.int32),
            pltpu.VMEM((CH, W), jnp.float32),
            pltpu.VMEM((CH, dh), jnp.float32),
            pltpu.VMEM((CH, dh), jnp.float32),
            pltpu.VMEM((CH, dh), jnp.float32),
            pltpu.VMEM_SHARED((n, dh), jnp.float32),
            pltpu.SemaphoreType.DMA,
            pltpu.SemaphoreType.DMA,
        ],
    )
    def k(hx_ref, w8_ref, subj_ref, objn_ref, zero_ref, out_ref,
          idxs, idxo, wb, buf, ms, mo, acc, sem, sem2):
        c = lax.axis_index("c")
        s = lax.axis_index("s")

        for p in range(2):
            c0 = c * 256 + p * dh
            _scatter_pass(hx_ref, w8_ref, subj_ref, objn_ref, zero_ref,
                          out_ref, idxs, idxo, wb, buf, ms, mo, acc, sem,
                          sem2, s, c0, n, nch, rows, dh)

    zeros = jnp.zeros((n // NS, dh), jnp.float32)
    return k(hx_edge, w8, subj, objn, zeros)


def _scatter_pass(hx_ref, w8_ref, subj_ref, objn_ref, zero_ref, out_ref,
                  idxs, idxo, wb, buf, ms, mo, acc, sem, sem2, s, c0, n,
                  nch, rows, dh):
        pltpu.sync_copy(zero_ref.at[pl.ds(0, rows), :],
                        acc.at[pl.ds(s * rows, rows), :])
        @pl.when(s == 0)
        def _():
            pltpu.sync_copy(zero_ref.at[pl.ds(0, n - rows * NS), :],
                            acc.at[pl.ds(rows * NS, n - rows * NS), :])
        plsc.subcore_barrier()

        def body(k_, _):
            e0 = (s + k_ * NS) * CH

            @pl.when(k_ > 0)
            def _():
                pltpu.make_async_copy(ms, acc.at[idxs], sem2).wait()
                pltpu.make_async_copy(mo, acc.at[idxo], sem2).wait()

            l1 = pltpu.async_copy(subj_ref.at[pl.ds(e0, CH)], idxs, sem)
            l2 = pltpu.async_copy(objn_ref.at[pl.ds(e0, CH)], idxo, sem)
            l3 = pltpu.async_copy(w8_ref.at[pl.ds(e0, CH), :], wb, sem)
            l4 = pltpu.async_copy(
                hx_ref.at[pl.ds(e0, CH), pl.ds(c0, dh)], buf, sem)
            l1.wait()
            l2.wait()
            l3.wait()
            l4.wait()

            def scale(j, _):
                wv = wb[j, :]
                ws = wv[0]
                wo = wv[1]
                for t in range(dh // 16):
                    v = buf[j, pl.ds(t * 16, 16)]
                    ms[j, pl.ds(t * 16, 16)] = v * ws
                    mo[j, pl.ds(t * 16, 16)] = v * wo
                return 0

            lax.fori_loop(0, CH, scale, 0)
            pltpu.async_copy(ms, acc.at[idxs], sem2, add=True)
            pltpu.async_copy(mo, acc.at[idxo], sem2, add=True)
            return 0

        trip = (nch - s + NS - 1) // NS
        lax.fori_loop(0, trip, body, 0)
        pltpu.make_async_copy(ms, acc.at[idxs], sem2).wait()
        pltpu.make_async_copy(mo, acc.at[idxo], sem2).wait()
        plsc.subcore_barrier()

        pltpu.sync_copy(acc.at[pl.ds(s * rows, rows), :],
                        out_ref.at[pl.ds(s * rows, rows), pl.ds(c0, dh)])
        @pl.when(s == 0)
        def _():
            pltpu.sync_copy(
                acc.at[pl.ds(rows * NS, n - rows * NS), :],
                out_ref.at[pl.ds(rows * NS, n - rows * NS), pl.ds(c0, dh)])
        plsc.subcore_barrier()


def kernel(x_obj, x_pred_feat, rel_inds, obj_W1, obj_b1, obj_W2, obj_b2,
           pred_W1, pred_b1, pred_W2, pred_b2, node_Wih, node_Whh, node_bih,
           node_bhh, edge_Wih, edge_Whh, edge_bih, edge_bhh, sn_W, sn_b,
           on_W, on_b, se_W, se_b, oe_W, oe_b, objcls_W, objcls_b, relcls_W,
           relcls_b):
    n = x_obj.shape[0]
    e = x_pred_feat.shape[0]
    d = obj_W2.shape[1]
    subj = rel_inds[:, 0]
    objn = rel_inds[:, 1]

    cnt = _sc_count(subj, objn, n)

    wg_n = jnp.concatenate(
        [sn_W[:d], on_W[:d], se_W[:d], oe_W[:d],
         jnp.zeros((d, 124), jnp.float32)], axis=1)
    wg_e = jnp.concatenate(
        [sn_W[d:], on_W[d:], se_W[d:], oe_W[d:],
         jnp.zeros((d, W - 4), jnp.float32)], axis=1)
    bg_e = jnp.concatenate([sn_b, on_b, se_b, oe_b,
                            jnp.zeros((W - 4,), jnp.float32)]).reshape(1, W)

    hx_edge, es = _mlp_edge(x_pred_feat, pred_W1, _row(pred_b1), pred_W2,
                            _row(pred_b2), wg_e, bg_e)
    hx_obj, tab = _mlp_node(x_obj, obj_W1, _row(obj_b1), obj_W2,
                            _row(obj_b2), wg_n, cnt)

    nbih, nbhh = _row(node_bih), _row(node_bhh)
    ebih, ebhh = _row(edge_bih), _row(edge_bhh)
    ocb, rcb = _row(objcls_b), _row(relcls_b)

    for _ in range(2):
        sub, obj, ts, to = _sc_gather(hx_obj, tab, subj, objn)
        w8 = _gates(ts, to, es)
        msg = _sc_scatter(hx_edge, w8, subj, objn, n)
        hx_obj, tab, obj_logits, labels8 = _node_step(
            msg, hx_obj, node_Wih, node_Whh, nbih, nbhh, wg_n, cnt,
            objcls_W, ocb)
        hx_edge, es, rel_logits = _edge_step(
            hx_edge, sub, obj, w8, edge_Wih, edge_Whh, ebih, ebhh, wg_e,
            bg_e, relcls_W, rcb)

    labels = labels8[:, 0]
    return ((hx_obj, hx_edge), obj_logits, rel_logits, labels, rel_inds)
